# baseline (speedup 1.0000x reference)
import functools

import jax
import jax.numpy as jnp
from jax import lax
from jax.experimental import pallas as pl
from jax.experimental.pallas import tpu as pltpu
from jax.experimental.pallas import tpu_sc as plsc

N_NODES = 10000
NP = 10240
N_EDGES = 320000
EP_ROWS = 2560
D_NODE = 128
D_EDGE = 16
NEG_SLOPE = 0.2

NC = 2
NS = 16
L = 16
NW = NC * NS
EPT = N_EDGES // NW
RPT = NP // NS
NE_P = NW * NP
ROWS_PT = NP // L
_UNROLL = 5

DH = D_NODE // 2
CH = 64
RPC = NE_P // CH // NS
NBUF = 4
NCH_IT = RPC // NBUF

_GRID = 10
_RB = NP // _GRID
_EB = EP_ROWS // _GRID

_MESH = plsc.VectorSubcoreMesh(core_axis_name="c", subcore_axis_name="s")
_SC_PARAMS = pltpu.CompilerParams(needs_layout_passes=False)
_SC_PARAMS_NT = pltpu.CompilerParams(needs_layout_passes=False,
                                     use_tc_tiling_on_sc=False)


def _prep_body(x_ref, w_ref, wsel_ref, ea_ref, m_ref, xw_ref, sp_ref, ew_ref):
    xw = jnp.dot(x_ref[...], w_ref[...], preferred_element_type=jnp.float32)
    xw_ref[...] = xw
    sp_ref[...] = lax.dot_general(wsel_ref[...], xw, (((0,), (1,)), ((), ())),
                                  preferred_element_type=jnp.float32)
    ew_ref[...] = jnp.dot(ea_ref[...], m_ref[...],
                          preferred_element_type=jnp.float32)


def _prep(x, w_node, wsel, ea2, m):
    return pl.pallas_call(
        _prep_body,
        grid=(_GRID,),
        in_specs=[
            pl.BlockSpec((_RB, D_NODE), lambda i: (i, 0)),
            pl.BlockSpec((D_NODE, D_NODE), lambda i: (0, 0)),
            pl.BlockSpec((D_NODE, 8), lambda i: (0, 0)),
            pl.BlockSpec((_EB, D_NODE * D_EDGE), lambda i: (i, 0)),
            pl.BlockSpec((D_NODE * D_EDGE, D_NODE), lambda i: (0, 0)),
        ],
        out_specs=[
            pl.BlockSpec((_RB, D_NODE), lambda i: (i, 0)),
            pl.BlockSpec((8, _RB), lambda i: (0, i)),
            pl.BlockSpec((_EB, D_NODE), lambda i: (i, 0)),
        ],
        out_shape=[
            jax.ShapeDtypeStruct((NP, D_NODE), jnp.float32),
            jax.ShapeDtypeStruct((8, NP), jnp.float32),
            jax.ShapeDtypeStruct((EP_ROWS, D_NODE), jnp.float32),
        ],
    )(x, w_node, wsel, ea2, m)


RPH = RPC // 2
EW_P = 32

@functools.partial(
    pl.kernel,
    out_type=(
        jax.ShapeDtypeStruct((NS, NP), jnp.float32),
        jax.ShapeDtypeStruct((NC * NP, DH), jnp.float32),
    ),
    mesh=_MESH,
    compiler_params=_SC_PARAMS_NT,
    scratch_types=[
        pltpu.VMEM((RPH, CH), jnp.int32),
        pltpu.VMEM((RPH, CH), jnp.int32),
        pltpu.VMEM((RPH, CH), jnp.float32),
        pltpu.VMEM((NP,), jnp.float32),
        pltpu.VMEM((NP,), jnp.float32),
        pltpu.VMEM((NP,), jnp.float32),
        pltpu.VMEM_SHARED((NP, DH), jnp.float32),
    ]
    + [pltpu.VMEM((CH, DH), jnp.float32) for _ in range(NBUF)]
    + [pltpu.SemaphoreType.DMA for _ in range(2 * NBUF)],
)
def _edge_scatter(spt_hbm, ei2_hbm, ew_hbm, xw2_hbm, den_hbm, acc_hbm,
                  src_v, dst_v, p_v, si_v, sj_v, den_v, acc_sh,
                  *bufs_and_sems):
    bufs = bufs_and_sems[:NBUF]
    gsems = bufs_and_sems[NBUF:2 * NBUF]
    ssems = bufs_and_sems[2 * NBUF:]

    cid = lax.axis_index("c")
    sid = lax.axis_index("s")
    zeros = jnp.zeros((L,), jnp.float32)

    pltpu.sync_copy(spt_hbm.at[pl.ds(0, NP)], si_v)
    pltpu.sync_copy(spt_hbm.at[pl.ds(NP, NP)], sj_v)

    def _zero(i, c):
        den_v[pl.ds(i * L, L)] = zeros
        return c

    lax.fori_loop(0, NP // L, _zero, 0)

    zbuf = bufs[0]

    def _zrow(r, c):
        for k in range(DH // L):
            zbuf[r, pl.ds(k * L, L)] = zeros
        return c

    lax.fori_loop(0, CH, _zrow, 0)
    for j in range(RPT // CH):
        pltpu.sync_copy(zbuf, acc_sh.at[pl.ds(sid * RPT + j * CH, CH)])

    plsc.subcore_barrier()

    hoff = jnp.full((L,), cid * NP, jnp.int32)
    ew_v = bufs[1]

    def _scale(buf, row):
        def _sgroup(g, c):
            pg = p_v[row, pl.ds(g * L, L)]
            for e in range(L):
                pv = jnp.full((L,), pg[e], jnp.float32)
                row16 = g * L + e
                for k in range(DH // L):
                    sl = pl.ds(k * L, L)
                    buf[row16, sl] = buf[row16, sl] * pv
            return c

        lax.fori_loop(0, CH // L, _sgroup, 0)

    for half in range(2):
        rbase = sid * RPC + half * RPH
        ebase = rbase * CH

        pltpu.sync_copy(ei2_hbm.at[pl.ds(rbase, RPH)], src_v)
        pltpu.sync_copy(ei2_hbm.at[pl.ds(NE_P // CH + rbase, RPH)], dst_v)

        for q in range(RPH // EW_P):
            pltpu.sync_copy(
                ew_hbm.at[pl.ds(rbase + q * EW_P, EW_P)],
                ew_v.at[pl.ds(0, EW_P)])

            def _row(r, c):
                row = q * EW_P + r
                real = (rbase + row) * CH < N_EDGES
                for g in range(CH // L):
                    sl = pl.ds(g * L, L)
                    s16 = src_v[row, sl]
                    d16 = dst_v[row, sl]
                    a = (plsc.load_gather(sj_v, [s16])
                         + plsc.load_gather(si_v, [d16])
                         + ew_v[r, sl])
                    a = jnp.where(a >= 0, a, NEG_SLOPE * a)
                    p = jnp.where(real, jnp.exp(a), zeros)
                    p_v[row, sl] = p
                    plsc.addupdate_scatter(den_v, [d16], p)
                return c

            lax.fori_loop(0, EW_P, _row, 0)

        def _adj(r, c):
            for k in range(CH // L):
                sl = pl.ds(k * L, L)
                src_v[r, sl] = src_v[r, sl] + hoff
            return c

        lax.fori_loop(0, RPH, _adj, 0)

        def _iter(ti, c):
            r0 = ti * NBUF
            gds = []
            for b in range(NBUF):
                gds.append(pltpu.async_copy(
                    xw2_hbm.at[src_v.at[r0 + b]], bufs[b], gsems[b]))
            sds = []
            for b in range(NBUF):
                gds[b].wait()
                _scale(bufs[b], r0 + b)
                sds.append(pltpu.async_copy(
                    bufs[b], acc_sh.at[dst_v.at[r0 + b]], ssems[b],
                    add=True))
            for b in range(NBUF):
                sds[b].wait()
            return c

        lax.fori_loop(0, RPH // NBUF, _iter, 0)

    @pl.when(cid == 0)
    def _():
        pltpu.sync_copy(den_v, den_hbm.at[sid])

    plsc.subcore_barrier()
    pltpu.sync_copy(acc_sh.at[pl.ds(sid * RPT, RPT)],
                    acc_hbm.at[pl.ds(cid * NP + sid * RPT, RPT)])


def _final_body(acc_ref, den_ref, b_ref, out_ref):
    t = jnp.concatenate([acc_ref[0], acc_ref[1]], axis=-1)
    den = jnp.sum(den_ref[...], axis=0)
    out_ref[...] = t / (den[:, None] + 1e-16) + b_ref[...]


def _final(acc3, denoms, bias2):
    return pl.pallas_call(
        _final_body,
        grid=(_GRID,),
        in_specs=[
            pl.BlockSpec((NC, _RB, DH), lambda i: (0, i, 0)),
            pl.BlockSpec((NS, _RB), lambda i: (0, i)),
            pl.BlockSpec((1, D_NODE), lambda i: (0, 0)),
        ],
        out_specs=pl.BlockSpec((_RB, D_NODE), lambda i: (i, 0)),
        out_shape=jax.ShapeDtypeStruct((NP, D_NODE), jnp.float32),
    )(acc3, denoms, bias2)


def kernel(x, edge_index, edge_attr, weight_node, weight_triplet_att, bias):
    ei32 = edge_index.astype(jnp.int32)

    watt = weight_triplet_att[0]
    w_i = watt[:D_NODE]
    w_e = watt[D_NODE:D_NODE + D_EDGE]
    w_j = watt[D_NODE + D_EDGE:]

    wsel = jnp.zeros((D_NODE, 8), jnp.float32).at[:, 0].set(w_i).at[:, 1].set(w_j)
    m = jnp.zeros((D_NODE * D_EDGE, D_NODE), jnp.float32).at[
        jnp.arange(D_NODE * D_EDGE),
        jnp.repeat(jnp.arange(D_NODE), D_EDGE)].set(jnp.tile(w_e, D_NODE))

    x_pad = jnp.pad(x, ((0, NP - N_NODES), (0, 0)))
    ea2 = jnp.pad(edge_attr.reshape(N_EDGES // D_NODE, D_NODE * D_EDGE),
                  ((0, EP_ROWS - N_EDGES // D_NODE), (0, 0)))

    xw, spt, ew2 = _prep(x_pad, weight_node, wsel, ea2, m)
    ew = ew2.reshape(NE_P)

    ei_pad = jnp.pad(ei32, ((0, 0), (0, NE_P - N_EDGES)))
    xw2 = jnp.concatenate([xw[:, :DH], xw[:, DH:]], axis=0)
    denoms, acc = _edge_scatter(spt.reshape(8 * NP),
                                ei_pad.reshape(2 * NE_P // CH, CH),
                                ew.reshape(NE_P // CH, CH), xw2)

    out = _final(acc.reshape(NC, NP, DH), denoms,
                 bias.reshape(1, D_NODE))
    return out[:N_NODES]


# speedup vs baseline: 1.1778x; 1.1778x over previous
import functools

import jax
import jax.numpy as jnp
from jax import lax
from jax.experimental import pallas as pl
from jax.experimental.pallas import tpu as pltpu
from jax.experimental.pallas import tpu_sc as plsc

N_NODES = 10000
NP = 10240
N_EDGES = 320000
EP_ROWS = 2560
D_NODE = 128
D_EDGE = 16
NEG_SLOPE = 0.2

NC = 2
NS = 16
L = 16
NW = NC * NS
EPT = N_EDGES // NW
RPT = NP // NS
NE_P = NW * NP
ROWS_PT = NP // L
_UNROLL = 5

DH = D_NODE // 2
CH = 64
RPC = NE_P // CH // NS
NBUF = 5
NCH_IT = RPC // NBUF

_GRID = 10
_RB = NP // _GRID
_EB = EP_ROWS // _GRID

_MESH = plsc.VectorSubcoreMesh(core_axis_name="c", subcore_axis_name="s")
_SC_PARAMS = pltpu.CompilerParams(needs_layout_passes=False)
_SC_PARAMS_NT = pltpu.CompilerParams(needs_layout_passes=False,
                                     use_tc_tiling_on_sc=False)


def _prep_body(x_ref, w_ref, wsel_ref, ea_ref, m_ref, xw_ref, sp_ref, ew_ref):
    xw = jnp.dot(x_ref[...], w_ref[...], preferred_element_type=jnp.float32)
    xw_ref[...] = xw
    sp_ref[...] = lax.dot_general(wsel_ref[...], xw, (((0,), (1,)), ((), ())),
                                  preferred_element_type=jnp.float32)
    ew_ref[...] = jnp.dot(ea_ref[...], m_ref[...],
                          preferred_element_type=jnp.float32)


def _prep(x, w_node, wsel, ea2, m):
    return pl.pallas_call(
        _prep_body,
        grid=(_GRID,),
        in_specs=[
            pl.BlockSpec((_RB, D_NODE), lambda i: (i, 0)),
            pl.BlockSpec((D_NODE, D_NODE), lambda i: (0, 0)),
            pl.BlockSpec((D_NODE, 8), lambda i: (0, 0)),
            pl.BlockSpec((_EB, D_NODE * D_EDGE), lambda i: (i, 0)),
            pl.BlockSpec((D_NODE * D_EDGE, D_NODE), lambda i: (0, 0)),
        ],
        out_specs=[
            pl.BlockSpec((_RB, D_NODE), lambda i: (i, 0)),
            pl.BlockSpec((8, _RB), lambda i: (0, i)),
            pl.BlockSpec((_EB, D_NODE), lambda i: (i, 0)),
        ],
        out_shape=[
            jax.ShapeDtypeStruct((NP, D_NODE), jnp.float32),
            jax.ShapeDtypeStruct((8, NP), jnp.float32),
            jax.ShapeDtypeStruct((EP_ROWS, D_NODE), jnp.float32),
        ],
    )(x, w_node, wsel, ea2, m)


@functools.partial(
    pl.kernel,
    out_type=(
        jax.ShapeDtypeStruct((N_EDGES,), jnp.float32),
        jax.ShapeDtypeStruct((NW, NP), jnp.float32),
    ),
    mesh=_MESH,
    compiler_params=_SC_PARAMS,
    scratch_types=[
        pltpu.VMEM((NP,), jnp.float32),
        pltpu.VMEM((NP,), jnp.float32),
        pltpu.VMEM((EPT,), jnp.int32),
        pltpu.VMEM((EPT,), jnp.int32),
        pltpu.VMEM((EPT,), jnp.float32),
        pltpu.VMEM((EPT,), jnp.float32),
        pltpu.VMEM((NP,), jnp.float32),
    ],
)
def _edge_softmax(spt_hbm, ei_hbm, ew_hbm, p_hbm, den_hbm,
                  si_v, sj_v, src_v, dst_v, ew_v, p_v, den_v):
    wid = lax.axis_index("s") * NC + lax.axis_index("c")
    base = wid * EPT
    pltpu.sync_copy(spt_hbm.at[pl.ds(0, NP)], si_v)
    pltpu.sync_copy(spt_hbm.at[pl.ds(NP, NP)], sj_v)
    pltpu.sync_copy(ei_hbm.at[pl.ds(base, EPT)], src_v)
    pltpu.sync_copy(ei_hbm.at[pl.ds(N_EDGES + base, EPT)], dst_v)
    pltpu.sync_copy(ew_hbm.at[pl.ds(base, EPT)], ew_v)

    zeros = jnp.zeros((L,), jnp.float32)

    def _zero(i, c):
        den_v[pl.ds(i * L, L)] = zeros
        return c

    lax.fori_loop(0, NP // L, _zero, 0)

    def _edge(it, c):
        for u in range(_UNROLL):
            off = (it * _UNROLL + u) * L
            s16 = src_v[pl.ds(off, L)]
            d16 = dst_v[pl.ds(off, L)]
            a = (plsc.load_gather(sj_v, [s16]) + plsc.load_gather(si_v, [d16])
                 + ew_v[pl.ds(off, L)])
            a = jnp.where(a >= 0, a, NEG_SLOPE * a)
            p = jnp.exp(a)
            p_v[pl.ds(off, L)] = p
            plsc.addupdate_scatter(den_v, [d16], p)
        return c

    lax.fori_loop(0, EPT // L // _UNROLL, _edge, 0)

    pltpu.sync_copy(p_v, p_hbm.at[pl.ds(base, EPT)])
    pltpu.sync_copy(den_v, den_hbm.at[wid])


@functools.partial(
    pl.kernel,
    out_type=jax.ShapeDtypeStruct((NC * NP, DH), jnp.float32),
    mesh=_MESH,
    compiler_params=_SC_PARAMS_NT,
    scratch_types=[
        pltpu.VMEM((RPC, CH), jnp.int32),
        pltpu.VMEM((RPC, CH), jnp.int32),
        pltpu.VMEM((RPC, CH), jnp.float32),
        pltpu.VMEM_SHARED((NP, DH), jnp.float32),
    ]
    + [pltpu.VMEM((CH, DH), jnp.float32) for _ in range(NBUF)]
    + [pltpu.SemaphoreType.DMA for _ in range(2 * NBUF)],
)
def _scatter(xw2_hbm, ei2_hbm, p2_hbm, acc_hbm,
             src_v, dst_v, p_v, acc_sh, *bufs_and_sems):
    bufs = bufs_and_sems[:NBUF]
    gsems = bufs_and_sems[NBUF:2 * NBUF]
    ssems = bufs_and_sems[2 * NBUF:]

    cid = lax.axis_index("c")
    sid = lax.axis_index("s")
    rbase = sid * RPC

    pltpu.sync_copy(ei2_hbm.at[pl.ds(rbase, RPC)], src_v)
    pltpu.sync_copy(ei2_hbm.at[pl.ds(NE_P // CH + rbase, RPC)], dst_v)
    pltpu.sync_copy(p2_hbm.at[pl.ds(rbase, RPC)], p_v)

    hoff = jnp.full((L,), cid * NP, jnp.int32)

    def _adj(r, c):
        for k in range(CH // L):
            sl = pl.ds(k * L, L)
            src_v[r, sl] = src_v[r, sl] + hoff
        return c

    lax.fori_loop(0, RPC, _adj, 0)

    zeros = jnp.zeros((L,), jnp.float32)
    zbuf = bufs[0]

    def _zrow(r, c):
        for k in range(DH // L):
            zbuf[r, pl.ds(k * L, L)] = zeros
        return c

    lax.fori_loop(0, CH, _zrow, 0)
    for j in range(RPT // CH):
        pltpu.sync_copy(zbuf, acc_sh.at[pl.ds(sid * RPT + j * CH, CH)])

    plsc.subcore_barrier()

    def _scale(buf, row):
        def _sgroup(g, c):
            pg = p_v[row, pl.ds(g * L, L)]
            for e in range(L):
                pv = jnp.full((L,), pg[e], jnp.float32)
                row16 = g * L + e
                for k in range(DH // L):
                    sl = pl.ds(k * L, L)
                    buf[row16, sl] = buf[row16, sl] * pv
            return c

        lax.fori_loop(0, CH // L, _sgroup, 0)

    def _iter(ti, c):
        r0 = ti * NBUF
        gds = []
        for b in range(NBUF):
            gds.append(pltpu.async_copy(
                xw2_hbm.at[src_v.at[r0 + b]], bufs[b], gsems[b]))
        sds = []
        for b in range(NBUF):
            gds[b].wait()
            _scale(bufs[b], r0 + b)
            sds.append(pltpu.async_copy(
                bufs[b], acc_sh.at[dst_v.at[r0 + b]], ssems[b], add=True))
        for b in range(NBUF):
            sds[b].wait()
        return c

    lax.fori_loop(0, NCH_IT, _iter, 0)

    plsc.subcore_barrier()
    pltpu.sync_copy(acc_sh.at[pl.ds(sid * RPT, RPT)],
                    acc_hbm.at[pl.ds(cid * NP + sid * RPT, RPT)])


def _final_body(acc_ref, den_ref, b_ref, out_ref):
    t = jnp.concatenate([acc_ref[0], acc_ref[1]], axis=-1)
    den = jnp.sum(den_ref[...], axis=0)
    out_ref[...] = t / (den[:, None] + 1e-16) + b_ref[...]


def _final(acc3, denoms, bias2):
    return pl.pallas_call(
        _final_body,
        grid=(_GRID,),
        in_specs=[
            pl.BlockSpec((NC, _RB, DH), lambda i: (0, i, 0)),
            pl.BlockSpec((NW, _RB), lambda i: (0, i)),
            pl.BlockSpec((1, D_NODE), lambda i: (0, 0)),
        ],
        out_specs=pl.BlockSpec((_RB, D_NODE), lambda i: (i, 0)),
        out_shape=jax.ShapeDtypeStruct((NP, D_NODE), jnp.float32),
    )(acc3, denoms, bias2)


def kernel(x, edge_index, edge_attr, weight_node, weight_triplet_att, bias):
    ei32 = edge_index.astype(jnp.int32)

    watt = weight_triplet_att[0]
    w_i = watt[:D_NODE]
    w_e = watt[D_NODE:D_NODE + D_EDGE]
    w_j = watt[D_NODE + D_EDGE:]

    wsel = jnp.zeros((D_NODE, 8), jnp.float32).at[:, 0].set(w_i).at[:, 1].set(w_j)
    m = jnp.zeros((D_NODE * D_EDGE, D_NODE), jnp.float32).at[
        jnp.arange(D_NODE * D_EDGE),
        jnp.repeat(jnp.arange(D_NODE), D_EDGE)].set(jnp.tile(w_e, D_NODE))

    x_pad = jnp.pad(x, ((0, NP - N_NODES), (0, 0)))
    ea2 = jnp.pad(edge_attr.reshape(N_EDGES // D_NODE, D_NODE * D_EDGE),
                  ((0, EP_ROWS - N_EDGES // D_NODE), (0, 0)))

    xw, spt, ew2 = _prep(x_pad, weight_node, wsel, ea2, m)
    ew = ew2.reshape(EP_ROWS * D_NODE)

    p, denoms = _edge_softmax(spt.reshape(8 * NP), ei32.reshape(2 * N_EDGES),
                              ew)

    ei_pad = jnp.pad(ei32, ((0, 0), (0, NE_P - N_EDGES)))
    p_pad = jnp.pad(p, (0, NE_P - N_EDGES))
    xw2 = jnp.concatenate([xw[:, :DH], xw[:, DH:]], axis=0)
    acc = _scatter(xw2, ei_pad.reshape(2 * NE_P // CH, CH),
                   p_pad.reshape(NE_P // CH, CH))

    out = _final(acc.reshape(NC, NP, DH), denoms,
                 bias.reshape(1, D_NODE))
    return out[:N_NODES]
